# Initial kernel scaffold; baseline (speedup 1.0000x reference)
import jax
import jax.numpy as jnp
from jax.experimental import pallas as pl


def kernel(x, params):
    raise NotImplementedError("write your pallas kernel here")


# speedup vs baseline: 19.3129x; 19.3129x over previous
import functools

import jax
import jax.numpy as jnp
from jax import lax
from jax.experimental import pallas as pl
from jax.experimental.pallas import tpu as pltpu
from jax.experimental.pallas import tpu_sc as plsc

F32 = jnp.float32
I32 = jnp.int32
B = 8
NPTS = 1024
NW = 32


def _mxdot(a, b):
    return lax.dot_general(a.astype(jnp.bfloat16), b.astype(jnp.bfloat16),
                           (((1,), (0,)), ((), ())),
                           preferred_element_type=F32)


def _fps_call(xc, yc, zc, npoint):
    Bb, N = xc.shape

    def body(x_ref, y_ref, z_ref, ox_ref, oy_ref, oz_ref):
        xx = x_ref[...]
        yy = y_ref[...]
        zz = z_ref[...]
        iota = lax.broadcasted_iota(I32, (Bb, N), 1)
        iota_s = lax.broadcasted_iota(I32, (Bb, npoint), 1)

        def it(i, st):
            dist, far = st
            oh = iota == far
            cx = jnp.sum(jnp.where(oh, xx, 0.0), 1, keepdims=True)
            cy = jnp.sum(jnp.where(oh, yy, 0.0), 1, keepdims=True)
            cz = jnp.sum(jnp.where(oh, zz, 0.0), 1, keepdims=True)
            ohs = iota_s == i
            ox_ref[...] = jnp.where(ohs, cx, ox_ref[...])
            oy_ref[...] = jnp.where(ohs, cy, oy_ref[...])
            oz_ref[...] = jnp.where(ohs, cz, oz_ref[...])
            d = (xx - cx) ** 2 + (yy - cy) ** 2 + (zz - cz) ** 2
            dist = jnp.minimum(dist, d)
            m = jnp.max(dist, 1, keepdims=True)
            far = jnp.min(jnp.where(dist == m, iota, N), 1, keepdims=True)
            return dist, far.astype(I32)

        lax.fori_loop(
            0, npoint, it,
            (jnp.full((Bb, N), 1e10, F32), jnp.zeros((Bb, 1), I32)),
        )

    return pl.pallas_call(
        body,
        out_shape=[jax.ShapeDtypeStruct((Bb, npoint), F32)] * 3,
    )(xc, yc, zc)


def _sqdist_call(a8, t8):
    Bb, S, _ = a8.shape
    N = t8.shape[2]

    def body(a_ref, t_ref, o_ref):
        a = a_ref[0]
        t = t_ref[0]
        dot = _mxdot(a, t)
        sa = jnp.sum(a * a, 1, keepdims=True)
        sb = jnp.sum(t * t, 0, keepdims=True)
        o_ref[0] = (sa + sb) - 2.0 * dot

    return pl.pallas_call(
        body,
        grid=(Bb,),
        in_specs=[
            pl.BlockSpec((1, S, 8), lambda b: (b, 0, 0)),
            pl.BlockSpec((1, 8, N), lambda b: (b, 0, 0)),
        ],
        out_specs=pl.BlockSpec((1, S, N), lambda b: (b, 0, 0)),
        out_shape=jax.ShapeDtypeStruct((Bb, S, N), F32),
    )(a8, t8)


def _ballq_call(dflat, S, N, radii, nss):
    BS = dflat.shape[0]
    rpw = BS // NW
    nb = N // 16
    RB = min(rpw, 32)
    NBLK = rpw // RB
    r2s = [float(r) * float(r) for r in radii]
    mesh = plsc.VectorSubcoreMesh(core_axis_name="c", subcore_axis_name="s")
    scratch = [pltpu.VMEM((RB, N), F32)]
    scratch += [pltpu.VMEM((N + 16,), I32) for _ in nss]
    scratch += [pltpu.VMEM((RB, ns), I32) for ns in nss]

    @functools.partial(
        pl.kernel,
        out_type=tuple(jax.ShapeDtypeStruct((BS, ns), I32) for ns in nss),
        mesh=mesh,
        compiler_params=pltpu.CompilerParams(needs_layout_passes=False),
        scratch_types=scratch,
    )
    def k(d_hbm, o1, o2, o3, dblk, b1, b2, b3, s1, s2, s3):
        wid = lax.axis_index("s") * 2 + lax.axis_index("c")
        bufs = (b1, b2, b3)
        stgs = (s1, s2, s3)
        outs = (o1, o2, o3)

        for blk in range(NBLK):
            def row(jj, carry, blk=blk):
                r = wid * rpw + blk * RB + jj
                base = (r // S) * N

                def chunk(i, cnts):
                    d = dblk[jj, pl.ds(i * 16, 16)]
                    gi = lax.iota(I32, 16) + (i * 16 + base)
                    nxt = []
                    for t in range(3):
                        m = d <= r2s[t]
                        pref = plsc.cumsum(m.astype(I32))
                        pos = cnts[t] + pref - 1
                        plsc.store_scatter(bufs[t], [pos], gi, mask=m)
                        nxt.append(cnts[t] + jnp.sum(m.astype(I32)))
                    return tuple(nxt)

                z = jnp.zeros((), I32)
                cnts = lax.fori_loop(0, nb + 0 * wid, chunk, (z, z, z))
                for t in range(3):
                    def fix(kk, c, t=t):
                        pos = lax.iota(I32, 16) + kk * 16
                        qpos = jnp.where(pos < cnts[t], pos, 0)
                        val = plsc.load_gather(bufs[t], [qpos])
                        val = jnp.where(cnts[t] == 0, base + (N - 1), val)
                        stgs[t][jj, pl.ds(kk * 16, 16)] = val
                        return c

                    lax.fori_loop(0, nss[t] // 16, fix, z)
                return carry

            row0 = wid * rpw + blk * RB
            pltpu.sync_copy(d_hbm.at[pl.ds(row0, RB)], dblk)
            lax.fori_loop(0, RB + 0 * wid, row, jnp.zeros((), I32))
            for t in range(3):
                pltpu.sync_copy(stgs[t], outs[t].at[pl.ds(row0, RB)])

    return k(dflat)


def _gather_call(tab, idx, C):
    R = idx.shape[0]
    G = 128
    rpw = R // NW
    nch = rpw // G
    mesh = plsc.VectorSubcoreMesh(core_axis_name="c", subcore_axis_name="s")

    @functools.partial(
        pl.kernel,
        out_type=jax.ShapeDtypeStruct((R, C), F32),
        mesh=mesh,
        compiler_params=pltpu.CompilerParams(use_tc_tiling_on_sc=False),
        scratch_types=[
            pltpu.VMEM((G,), I32),
            pltpu.VMEM((G, C), F32),
            pltpu.SemaphoreType.DMA,
        ],
    )
    def k(tab_hbm, idx_hbm, out_hbm, idx_v, rows_v, sem):
        wid = lax.axis_index("s") * 2 + lax.axis_index("c")

        def chunk(i, c):
            base = wid * rpw + i * G
            pltpu.sync_copy(idx_hbm.at[pl.ds(base, G)], idx_v)
            pltpu.async_copy(tab_hbm.at[idx_v], rows_v, sem).wait()
            pltpu.sync_copy(rows_v, out_hbm.at[pl.ds(base, G)])
            return c

        lax.fori_loop(0, nch + 0 * wid, chunk, jnp.zeros((), I32))

    return k(tab, idx)


def _pool_call(g3, nx8, w1, bias1, w2, bias2, w3, bias3, sa1_mode, P):
    BS, ns, C = g3.shape
    c3 = w3.shape[1]

    def body(g_ref, nx_ref, w1_ref, b1_ref, w2_ref, b2_ref, w3_ref, b3_ref,
             o_ref):
        g = g_ref[...]
        nx = nx_ref[...]
        if sa1_mode:
            gr = (g - nx[:, None, :]).reshape(P * ns, C)
            h = jnp.maximum(
                _mxdot(gr, w1_ref[...])
                + b1_ref[...], 0.0)
        else:
            v = _mxdot(nx, w1_ref[...])
            h1 = jnp.maximum(g - v[:, None, :] + b1_ref[...][None], 0.0)
            h = h1.reshape(P * ns, C)
        h = jnp.maximum(
            _mxdot(h, w2_ref[...]) + b2_ref[...],
            0.0)
        h = jnp.maximum(
            _mxdot(h, w3_ref[...]) + b3_ref[...],
            0.0)
        o_ref[...] = jnp.max(h.reshape(P, ns, c3), axis=1)

    grid = BS // P
    full = lambda shp: pl.BlockSpec(shp, lambda i: tuple(0 for _ in shp))
    return pl.pallas_call(
        body,
        grid=(grid,),
        in_specs=[
            pl.BlockSpec((P, ns, C), lambda i: (i, 0, 0)),
            pl.BlockSpec((P, 8), lambda i: (i, 0)),
            full(w1.shape), full(bias1.shape), full(w2.shape),
            full(bias2.shape), full(w3.shape), full(bias3.shape),
        ],
        out_specs=pl.BlockSpec((P, c3), lambda i: (i, 0)),
        out_shape=jax.ShapeDtypeStruct((BS, c3), F32),
    )(g3, nx8, w1, bias1, w2, bias2, w3, bias3)


def _u2_call(xpad, w1, w2, w3):
    def body(x_ref, wa_ref, wb_ref, wc_ref, oa_ref, ob_ref, oc_ref):
        xx = x_ref[...]
        oa_ref[...] = _mxdot(xx, wa_ref[...])
        ob_ref[...] = _mxdot(xx, wb_ref[...])
        oc_ref[...] = _mxdot(xx, wc_ref[...])

    n = xpad.shape[0]
    return pl.pallas_call(
        body,
        out_shape=[
            jax.ShapeDtypeStruct((n, w1.shape[1]), F32),
            jax.ShapeDtypeStruct((n, w2.shape[1]), F32),
            jax.ShapeDtypeStruct((n, w3.shape[1]), F32),
        ],
    )(xpad, w1, w2, w3)


def _tail_call(xp, weights):
    def body(x_ref, w1_ref, b1_ref, w2_ref, b2_ref, w3_ref, b3_ref,
             wl1_ref, bl1_ref, wl2_ref, bl2_ref, wd1_ref, bd1_ref,
             wd2_ref, bd2_ref, wd3_ref, bd3_ref, o_ref):
        h = jnp.maximum(
            _mxdot(x_ref[...], w1_ref[...])
            + b1_ref[...], 0.0)
        h = jnp.maximum(
            _mxdot(h, w2_ref[...]) + b2_ref[...],
            0.0)
        h = jnp.maximum(
            _mxdot(h, w3_ref[...]) + b3_ref[...],
            0.0)
        t = jnp.max(h.reshape(B, 128, 1024), axis=1)
        h = jnp.maximum(
            _mxdot(t, wl1_ref[...])
            + bl1_ref[...], 0.0)
        e = _mxdot(h, wl2_ref[...]) + bl2_ref[...]
        h = jnp.maximum(
            _mxdot(e, wd1_ref[...])
            + bd1_ref[...], 0.0)
        h = jnp.maximum(
            _mxdot(h, wd2_ref[...])
            + bd2_ref[...], 0.0)
        o_ref[...] = (_mxdot(h, wd3_ref[...])
                      + bd3_ref[...])

    return pl.pallas_call(
        body,
        out_shape=jax.ShapeDtypeStruct((B, 3 * NPTS), F32),
    )(xp, *weights)


def _row_bias(b):
    return b.reshape(1, -1)


def kernel(x, params):
    xt = x.transpose(0, 2, 1)

    ox, oy, oz = _fps_call(xt[:, 0], xt[:, 1], xt[:, 2], 512)
    nx1 = jnp.stack([ox, oy, oz], axis=-1)
    a8 = jnp.pad(nx1, ((0, 0), (0, 0), (0, 5)))
    t8 = jnp.pad(xt, ((0, 0), (0, 5), (0, 0)))
    d1 = _sqdist_call(a8, t8).reshape(B * 512, NPTS)
    i1, i2, i3 = _ballq_call(d1, 512, NPTS, (0.1, 0.2, 0.4), (16, 32, 128))

    xyz_tab = jnp.pad(x.reshape(B * NPTS, 3), ((0, 0), (0, 5)))
    nx8flat = a8.reshape(B * 512, 8)
    outs1 = []
    for idx, ns, br, P in zip((i1, i2, i3), (16, 32, 128), params['sa1'],
                              (256, 128, 32)):
        gfl = _gather_call(xyz_tab, idx.reshape(-1), 8)
        g3 = gfl.reshape(B * 512, ns, 8)
        w1p = jnp.pad(br[0]['W'], ((0, 5), (0, 0)))
        pooled = _pool_call(
            g3, nx8flat, w1p, _row_bias(br[0]['b']),
            br[1]['W'], _row_bias(br[1]['b']),
            br[2]['W'], _row_bias(br[2]['b']),
            sa1_mode=True, P=P)
        outs1.append(pooled)
    l1_points = jnp.concatenate(outs1, axis=-1)

    xt2 = nx1.transpose(0, 2, 1)
    ox2, oy2, oz2 = _fps_call(xt2[:, 0], xt2[:, 1], xt2[:, 2], 128)
    nx2 = jnp.stack([ox2, oy2, oz2], axis=-1)
    a8_2 = jnp.pad(nx2, ((0, 0), (0, 0), (0, 5)))
    t8_2 = jnp.pad(xt2, ((0, 0), (0, 5), (0, 0)))
    d2 = _sqdist_call(a8_2, t8_2).reshape(B * 128, 512)
    j1, j2, j3 = _ballq_call(d2, 128, 512, (0.2, 0.4, 0.8), (32, 64, 128))

    cat2 = jnp.concatenate([l1_points.reshape(B, 512, 320), nx1], axis=-1)
    xpad = jnp.pad(cat2.reshape(B * 512, 323), ((0, 0), (0, 61)))
    w1s = [jnp.pad(br[0]['W'], ((0, 61), (0, 0))) for br in params['sa2']]
    u1, u2, u3 = _u2_call(xpad, *w1s)

    nx8f2 = a8_2.reshape(B * 128, 8)
    outs2 = []
    for idx, ns, br, u, P in zip((j1, j2, j3), (32, 64, 128), params['sa2'],
                                 (u1, u2, u3), (128, 64, 32)):
        c1 = br[0]['W'].shape[1]
        gfl = _gather_call(u, idx.reshape(-1), c1)
        g3 = gfl.reshape(B * 128, ns, c1)
        w1v = jnp.pad(br[0]['W'][320:, :], ((0, 5), (0, 0)))
        pooled = _pool_call(
            g3, nx8f2, w1v, _row_bias(br[0]['b']),
            br[1]['W'], _row_bias(br[1]['b']),
            br[2]['W'], _row_bias(br[2]['b']),
            sa1_mode=False, P=P)
        outs2.append(pooled)
    l2_points = jnp.concatenate(outs2, axis=-1).reshape(B, 128, 640)

    l2cat = jnp.concatenate([nx2, l2_points], axis=-1)
    xp = jnp.pad(l2cat.reshape(B * 128, 643), ((0, 0), (0, 125)))
    sa3 = params['sa3']
    weights = [
        jnp.pad(sa3[0]['W'], ((0, 125), (0, 0))), _row_bias(sa3[0]['b']),
        sa3[1]['W'], _row_bias(sa3[1]['b']),
        sa3[2]['W'], _row_bias(sa3[2]['b']),
        params['lin1']['W'], _row_bias(params['lin1']['b']),
        params['lin2']['W'], _row_bias(params['lin2']['b']),
        params['dec'][0]['W'], _row_bias(params['dec'][0]['b']),
        params['dec'][1]['W'], _row_bias(params['dec'][1]['b']),
        params['dec'][2]['W'], _row_bias(params['dec'][2]['b']),
    ]
    out = _tail_call(xp, weights)
    return out.reshape(B, 3, NPTS).transpose(0, 2, 1)


# speedup vs baseline: 21.6521x; 1.1211x over previous
import functools

import jax
import jax.numpy as jnp
from jax import lax
from jax.experimental import pallas as pl
from jax.experimental.pallas import tpu as pltpu
from jax.experimental.pallas import tpu_sc as plsc

F32 = jnp.float32
I32 = jnp.int32
B = 8
NPTS = 1024
NW = 32


def _mxdot(a, b):
    return lax.dot_general(a.astype(jnp.bfloat16), b.astype(jnp.bfloat16),
                           (((1,), (0,)), ((), ())),
                           preferred_element_type=F32)


def _fps_call(xc, yc, zc, npoint):
    Bb, N = xc.shape

    def body(x_ref, y_ref, z_ref, ox_ref, oy_ref, oz_ref):
        xx = x_ref[...]
        yy = y_ref[...]
        zz = z_ref[...]
        iota = lax.broadcasted_iota(I32, (Bb, N), 1)
        iota_s = lax.broadcasted_iota(I32, (Bb, npoint), 1)

        def it(i, st):
            dist, far = st
            oh = iota == far
            cx = jnp.sum(jnp.where(oh, xx, 0.0), 1, keepdims=True)
            cy = jnp.sum(jnp.where(oh, yy, 0.0), 1, keepdims=True)
            cz = jnp.sum(jnp.where(oh, zz, 0.0), 1, keepdims=True)
            ohs = iota_s == i
            ox_ref[...] = jnp.where(ohs, cx, ox_ref[...])
            oy_ref[...] = jnp.where(ohs, cy, oy_ref[...])
            oz_ref[...] = jnp.where(ohs, cz, oz_ref[...])
            d = (xx - cx) ** 2 + (yy - cy) ** 2 + (zz - cz) ** 2
            dist = jnp.minimum(dist, d)
            m = jnp.max(dist, 1, keepdims=True)
            far = jnp.min(jnp.where(dist == m, iota, N), 1, keepdims=True)
            return dist, far.astype(I32)

        lax.fori_loop(
            0, npoint, it,
            (jnp.full((Bb, N), 1e10, F32), jnp.zeros((Bb, 1), I32)),
        )

    return pl.pallas_call(
        body,
        out_shape=[jax.ShapeDtypeStruct((Bb, npoint), F32)] * 3,
    )(xc, yc, zc)


def _sqdist_call(a8, t8):
    Bb, S, _ = a8.shape
    N = t8.shape[2]

    def body(a_ref, t_ref, o_ref):
        a = a_ref[0]
        t = t_ref[0]
        dot = _mxdot(a, t)
        sa = jnp.sum(a * a, 1, keepdims=True)
        sb = jnp.sum(t * t, 0, keepdims=True)
        o_ref[0] = (sa + sb) - 2.0 * dot

    return pl.pallas_call(
        body,
        grid=(Bb,),
        in_specs=[
            pl.BlockSpec((1, S, 8), lambda b: (b, 0, 0)),
            pl.BlockSpec((1, 8, N), lambda b: (b, 0, 0)),
        ],
        out_specs=pl.BlockSpec((1, S, N), lambda b: (b, 0, 0)),
        out_shape=jax.ShapeDtypeStruct((Bb, S, N), F32),
    )(a8, t8)


def _ballq_call(dflat, S, N, radii, nss):
    BS = dflat.shape[0]
    rpw = BS // NW
    nb = N // 16
    RB = min(rpw, 32)
    NBLK = rpw // RB
    r2s = [float(r) * float(r) for r in radii]
    mesh = plsc.VectorSubcoreMesh(core_axis_name="c", subcore_axis_name="s")
    scratch = [pltpu.VMEM((RB, N), F32)]
    scratch += [pltpu.VMEM((N + 16,), I32) for _ in nss]
    scratch += [pltpu.VMEM((RB, ns), I32) for ns in nss]

    @functools.partial(
        pl.kernel,
        out_type=tuple(jax.ShapeDtypeStruct((BS, ns), I32) for ns in nss),
        mesh=mesh,
        compiler_params=pltpu.CompilerParams(needs_layout_passes=False),
        scratch_types=scratch,
    )
    def k(d_hbm, o1, o2, o3, dblk, b1, b2, b3, s1, s2, s3):
        wid = lax.axis_index("s") * 2 + lax.axis_index("c")
        bufs = (b1, b2, b3)
        stgs = (s1, s2, s3)
        outs = (o1, o2, o3)

        for blk in range(NBLK):
            def row(jj, carry, blk=blk):
                r = wid * rpw + blk * RB + jj
                base = (r // S) * N

                def chunk(i, cnts):
                    d = dblk[jj, pl.ds(i * 16, 16)]
                    gi = lax.iota(I32, 16) + (i * 16 + base)
                    nxt = []
                    for t in range(3):
                        m = d <= r2s[t]
                        pref = plsc.cumsum(m.astype(I32))
                        pos = cnts[t] + pref - 1
                        plsc.store_scatter(bufs[t], [pos], gi, mask=m)
                        nxt.append(cnts[t] + plsc.all_reduce_population_count(m))
                    return tuple(nxt)

                z = jnp.zeros((16,), I32)
                cnts = lax.fori_loop(0, nb + 0 * wid, chunk, (z, z, z))
                for t in range(3):
                    def fix(kk, c, t=t):
                        pos = lax.iota(I32, 16) + kk * 16
                        qpos = jnp.where(pos < cnts[t], pos, 0)
                        val = plsc.load_gather(bufs[t], [qpos])
                        val = jnp.where(cnts[t] == 0, base + (N - 1), val)
                        stgs[t][jj, pl.ds(kk * 16, 16)] = val
                        return c

                    lax.fori_loop(0, nss[t] // 16, fix, z)
                return carry

            row0 = wid * rpw + blk * RB
            pltpu.sync_copy(d_hbm.at[pl.ds(row0, RB)], dblk)
            lax.fori_loop(0, RB + 0 * wid, row, jnp.zeros((), I32))
            for t in range(3):
                pltpu.sync_copy(stgs[t], outs[t].at[pl.ds(row0, RB)])

    return k(dflat)


def _gather_call(tab, idx, C):
    R = idx.shape[0]
    G = 128
    K = 4
    rpw = R // NW
    nit = rpw // (G * K)
    idx2d = idx.reshape(R // G, G)
    mesh = plsc.VectorSubcoreMesh(core_axis_name="c", subcore_axis_name="s")

    @functools.partial(
        pl.kernel,
        out_type=jax.ShapeDtypeStruct((R, C), F32),
        mesh=mesh,
        compiler_params=pltpu.CompilerParams(use_tc_tiling_on_sc=False),
        scratch_types=[
            pltpu.VMEM((K, G), I32),
            pltpu.VMEM((K * G, C), F32),
            pltpu.SemaphoreType.DMA,
        ],
    )
    def k(tab_hbm, idx_hbm, out_hbm, idx_v, rows_v, sem):
        wid = lax.axis_index("s") * 2 + lax.axis_index("c")

        def chunk(i, c):
            base = wid * rpw + i * (G * K)
            pltpu.sync_copy(idx_hbm.at[pl.ds(base // G, K)], idx_v)
            cps = [
                pltpu.async_copy(tab_hbm.at[idx_v.at[j]],
                                 rows_v.at[pl.ds(j * G, G)], sem)
                for j in range(K)
            ]
            for cp in cps:
                cp.wait()
            pltpu.sync_copy(rows_v, out_hbm.at[pl.ds(base, G * K)])
            return c

        lax.fori_loop(0, nit + 0 * wid, chunk, jnp.zeros((), I32))

    return k(tab, idx2d)


def _pool_call(g3, nx8, w1, bias1, w2, bias2, w3, bias3, sa1_mode, P):
    BS, ns, C = g3.shape
    c3 = w3.shape[1]

    def body(g_ref, nx_ref, w1_ref, b1_ref, w2_ref, b2_ref, w3_ref, b3_ref,
             o_ref):
        g = g_ref[...]
        nx = nx_ref[...]
        if sa1_mode:
            gr = (g - nx[:, None, :]).reshape(P * ns, C)
            h = jnp.maximum(
                _mxdot(gr, w1_ref[...])
                + b1_ref[...], 0.0)
        else:
            v = _mxdot(nx, w1_ref[...])
            h1 = jnp.maximum(g - v[:, None, :] + b1_ref[...][None], 0.0)
            h = h1.reshape(P * ns, C)
        h = jnp.maximum(
            _mxdot(h, w2_ref[...]) + b2_ref[...],
            0.0)
        h = jnp.maximum(
            _mxdot(h, w3_ref[...]) + b3_ref[...],
            0.0)
        o_ref[...] = jnp.max(h.reshape(P, ns, c3), axis=1)

    grid = BS // P
    full = lambda shp: pl.BlockSpec(shp, lambda i: tuple(0 for _ in shp))
    return pl.pallas_call(
        body,
        grid=(grid,),
        in_specs=[
            pl.BlockSpec((P, ns, C), lambda i: (i, 0, 0)),
            pl.BlockSpec((P, 8), lambda i: (i, 0)),
            full(w1.shape), full(bias1.shape), full(w2.shape),
            full(bias2.shape), full(w3.shape), full(bias3.shape),
        ],
        out_specs=pl.BlockSpec((P, c3), lambda i: (i, 0)),
        out_shape=jax.ShapeDtypeStruct((BS, c3), F32),
    )(g3, nx8, w1, bias1, w2, bias2, w3, bias3)


def _u2_call(xpad, w1, w2, w3):
    def body(x_ref, wa_ref, wb_ref, wc_ref, oa_ref, ob_ref, oc_ref):
        xx = x_ref[...]
        oa_ref[...] = _mxdot(xx, wa_ref[...])
        ob_ref[...] = _mxdot(xx, wb_ref[...])
        oc_ref[...] = _mxdot(xx, wc_ref[...])

    n = xpad.shape[0]
    return pl.pallas_call(
        body,
        out_shape=[
            jax.ShapeDtypeStruct((n, w1.shape[1]), F32),
            jax.ShapeDtypeStruct((n, w2.shape[1]), F32),
            jax.ShapeDtypeStruct((n, w3.shape[1]), F32),
        ],
    )(xpad, w1, w2, w3)


def _tail_call(xp, weights):
    def body(x_ref, w1_ref, b1_ref, w2_ref, b2_ref, w3_ref, b3_ref,
             wl1_ref, bl1_ref, wl2_ref, bl2_ref, wd1_ref, bd1_ref,
             wd2_ref, bd2_ref, wd3_ref, bd3_ref, o_ref):
        h = jnp.maximum(
            _mxdot(x_ref[...], w1_ref[...])
            + b1_ref[...], 0.0)
        h = jnp.maximum(
            _mxdot(h, w2_ref[...]) + b2_ref[...],
            0.0)
        h = jnp.maximum(
            _mxdot(h, w3_ref[...]) + b3_ref[...],
            0.0)
        t = jnp.max(h.reshape(B, 128, 1024), axis=1)
        h = jnp.maximum(
            _mxdot(t, wl1_ref[...])
            + bl1_ref[...], 0.0)
        e = _mxdot(h, wl2_ref[...]) + bl2_ref[...]
        h = jnp.maximum(
            _mxdot(e, wd1_ref[...])
            + bd1_ref[...], 0.0)
        h = jnp.maximum(
            _mxdot(h, wd2_ref[...])
            + bd2_ref[...], 0.0)
        o_ref[...] = (_mxdot(h, wd3_ref[...])
                      + bd3_ref[...])

    return pl.pallas_call(
        body,
        out_shape=jax.ShapeDtypeStruct((B, 3 * NPTS), F32),
    )(xp, *weights)


def _row_bias(b):
    return b.reshape(1, -1)


def kernel(x, params):
    xt = x.transpose(0, 2, 1)

    ox, oy, oz = _fps_call(xt[:, 0], xt[:, 1], xt[:, 2], 512)
    nx1 = jnp.stack([ox, oy, oz], axis=-1)
    a8 = jnp.pad(nx1, ((0, 0), (0, 0), (0, 5)))
    t8 = jnp.pad(xt, ((0, 0), (0, 5), (0, 0)))
    d1 = _sqdist_call(a8, t8).reshape(B * 512, NPTS)
    i1, i2, i3 = _ballq_call(d1, 512, NPTS, (0.1, 0.2, 0.4), (16, 32, 128))

    xyz_tab = jnp.pad(x.reshape(B * NPTS, 3), ((0, 0), (0, 5)))
    nx8flat = a8.reshape(B * 512, 8)
    outs1 = []
    for idx, ns, br, P in zip((i1, i2, i3), (16, 32, 128), params['sa1'],
                              (256, 128, 32)):
        gfl = _gather_call(xyz_tab, idx.reshape(-1), 8)
        g3 = gfl.reshape(B * 512, ns, 8)
        w1p = jnp.pad(br[0]['W'], ((0, 5), (0, 0)))
        pooled = _pool_call(
            g3, nx8flat, w1p, _row_bias(br[0]['b']),
            br[1]['W'], _row_bias(br[1]['b']),
            br[2]['W'], _row_bias(br[2]['b']),
            sa1_mode=True, P=P)
        outs1.append(pooled)
    l1_points = jnp.concatenate(outs1, axis=-1)

    xt2 = nx1.transpose(0, 2, 1)
    ox2, oy2, oz2 = _fps_call(xt2[:, 0], xt2[:, 1], xt2[:, 2], 128)
    nx2 = jnp.stack([ox2, oy2, oz2], axis=-1)
    a8_2 = jnp.pad(nx2, ((0, 0), (0, 0), (0, 5)))
    t8_2 = jnp.pad(xt2, ((0, 0), (0, 5), (0, 0)))
    d2 = _sqdist_call(a8_2, t8_2).reshape(B * 128, 512)
    j1, j2, j3 = _ballq_call(d2, 128, 512, (0.2, 0.4, 0.8), (32, 64, 128))

    cat2 = jnp.concatenate([l1_points.reshape(B, 512, 320), nx1], axis=-1)
    xpad = jnp.pad(cat2.reshape(B * 512, 323), ((0, 0), (0, 61)))
    w1s = [jnp.pad(br[0]['W'], ((0, 61), (0, 0))) for br in params['sa2']]
    u1, u2, u3 = _u2_call(xpad, *w1s)

    nx8f2 = a8_2.reshape(B * 128, 8)
    outs2 = []
    for idx, ns, br, u, P in zip((j1, j2, j3), (32, 64, 128), params['sa2'],
                                 (u1, u2, u3), (128, 64, 32)):
        c1 = br[0]['W'].shape[1]
        gfl = _gather_call(u, idx.reshape(-1), c1)
        g3 = gfl.reshape(B * 128, ns, c1)
        w1v = jnp.pad(br[0]['W'][320:, :], ((0, 5), (0, 0)))
        pooled = _pool_call(
            g3, nx8f2, w1v, _row_bias(br[0]['b']),
            br[1]['W'], _row_bias(br[1]['b']),
            br[2]['W'], _row_bias(br[2]['b']),
            sa1_mode=False, P=P)
        outs2.append(pooled)
    l2_points = jnp.concatenate(outs2, axis=-1).reshape(B, 128, 640)

    l2cat = jnp.concatenate([nx2, l2_points], axis=-1)
    xp = jnp.pad(l2cat.reshape(B * 128, 643), ((0, 0), (0, 125)))
    sa3 = params['sa3']
    weights = [
        jnp.pad(sa3[0]['W'], ((0, 125), (0, 0))), _row_bias(sa3[0]['b']),
        sa3[1]['W'], _row_bias(sa3[1]['b']),
        sa3[2]['W'], _row_bias(sa3[2]['b']),
        params['lin1']['W'], _row_bias(params['lin1']['b']),
        params['lin2']['W'], _row_bias(params['lin2']['b']),
        params['dec'][0]['W'], _row_bias(params['dec'][0]['b']),
        params['dec'][1]['W'], _row_bias(params['dec'][1]['b']),
        params['dec'][2]['W'], _row_bias(params['dec'][2]['b']),
    ]
    out = _tail_call(xp, weights)
    return out.reshape(B, 3, NPTS).transpose(0, 2, 1)


# speedup vs baseline: 211.0000x; 9.7450x over previous
import functools

import jax
import jax.numpy as jnp
from jax import lax
from jax.experimental import pallas as pl
from jax.experimental.pallas import tpu as pltpu
from jax.experimental.pallas import tpu_sc as plsc

F32 = jnp.float32
I32 = jnp.int32
B = 8
NPTS = 1024
NW = 32


def _mxdot(a, b):
    return lax.dot_general(a.astype(jnp.bfloat16), b.astype(jnp.bfloat16),
                           (((1,), (0,)), ((), ())),
                           preferred_element_type=F32)


def _fps_call(xc, yc, zc, npoint):
    Bb, N = xc.shape

    def body(x_ref, y_ref, z_ref, ox_ref, oy_ref, oz_ref):
        xx = x_ref[...]
        yy = y_ref[...]
        zz = z_ref[...]
        iota = lax.broadcasted_iota(I32, (Bb, N), 1)
        iota_s = lax.broadcasted_iota(I32, (Bb, npoint), 1)

        def it(i, st):
            dist, far = st
            oh = iota == far
            cx = jnp.sum(jnp.where(oh, xx, 0.0), 1, keepdims=True)
            cy = jnp.sum(jnp.where(oh, yy, 0.0), 1, keepdims=True)
            cz = jnp.sum(jnp.where(oh, zz, 0.0), 1, keepdims=True)
            ohs = iota_s == i
            ox_ref[...] = jnp.where(ohs, cx, ox_ref[...])
            oy_ref[...] = jnp.where(ohs, cy, oy_ref[...])
            oz_ref[...] = jnp.where(ohs, cz, oz_ref[...])
            d = (xx - cx) ** 2 + (yy - cy) ** 2 + (zz - cz) ** 2
            dist = jnp.minimum(dist, d)
            m = jnp.max(dist, 1, keepdims=True)
            far = jnp.min(jnp.where(dist == m, iota, N), 1, keepdims=True)
            return dist, far.astype(I32)

        lax.fori_loop(
            0, npoint, it,
            (jnp.full((Bb, N), 1e10, F32), jnp.zeros((Bb, 1), I32)),
        )

    return pl.pallas_call(
        body,
        out_shape=[jax.ShapeDtypeStruct((Bb, npoint), F32)] * 3,
    )(xc, yc, zc)


def _sqdist_call(a8, t8):
    Bb, S, _ = a8.shape
    N = t8.shape[2]

    def body(a_ref, t_ref, o_ref):
        a = a_ref[0]
        t = t_ref[0]
        dot = _mxdot(a, t)
        sa = jnp.sum(a * a, 1, keepdims=True)
        sb = jnp.sum(t * t, 0, keepdims=True)
        o_ref[0] = (sa + sb) - 2.0 * dot

    return pl.pallas_call(
        body,
        grid=(Bb,),
        in_specs=[
            pl.BlockSpec((1, S, 8), lambda b: (b, 0, 0)),
            pl.BlockSpec((1, 8, N), lambda b: (b, 0, 0)),
        ],
        out_specs=pl.BlockSpec((1, S, N), lambda b: (b, 0, 0)),
        out_shape=jax.ShapeDtypeStruct((Bb, S, N), F32),
    )(a8, t8)


def _ballq_call(dflat, S, N, radii, nss):
    BS = dflat.shape[0]
    rpw = BS // NW
    nb = N // 16
    RB = min(rpw, 32)
    NBLK = rpw // RB
    r2s = [float(r) * float(r) for r in radii]
    mesh = plsc.VectorSubcoreMesh(core_axis_name="c", subcore_axis_name="s")
    scratch = [pltpu.VMEM((RB, N), F32)]
    scratch += [pltpu.VMEM((N + 16,), I32) for _ in nss]
    scratch += [pltpu.VMEM((RB, ns), I32) for ns in nss]

    @functools.partial(
        pl.kernel,
        out_type=tuple(jax.ShapeDtypeStruct((BS, ns), I32) for ns in nss),
        mesh=mesh,
        compiler_params=pltpu.CompilerParams(needs_layout_passes=False),
        scratch_types=scratch,
    )
    def k(d_hbm, o1, o2, o3, dblk, b1, b2, b3, s1, s2, s3):
        wid = lax.axis_index("s") * 2 + lax.axis_index("c")
        bufs = (b1, b2, b3)
        stgs = (s1, s2, s3)
        outs = (o1, o2, o3)

        for blk in range(NBLK):
            def row(jj, carry, blk=blk):
                r = wid * rpw + blk * RB + jj
                base = (r // S) * N

                def chunk(i, cnts):
                    for u in range(4):
                        ii = i * 4 + u
                        d = dblk[jj, pl.ds(ii * 16, 16)]
                        gi = lax.iota(I32, 16) + (ii * 16 + base)
                        nxt = []
                        for t in range(3):
                            m = d <= r2s[t]
                            pref = plsc.cumsum(m.astype(I32))
                            pos = cnts[t] + pref - 1
                            plsc.store_scatter(bufs[t], [pos], gi, mask=m)
                            nxt.append(
                                cnts[t] + plsc.all_reduce_population_count(m))
                        cnts = tuple(nxt)
                    return cnts

                z = jnp.zeros((16,), I32)
                cnts = lax.fori_loop(0, nb // 4 + 0 * wid, chunk, (z, z, z))
                for t in range(3):
                    def fix(kk, c, t=t):
                        pos = lax.iota(I32, 16) + kk * 16
                        qpos = jnp.where(pos < cnts[t], pos, 0)
                        val = plsc.load_gather(bufs[t], [qpos])
                        val = jnp.where(cnts[t] == 0, base + (N - 1), val)
                        stgs[t][jj, pl.ds(kk * 16, 16)] = val
                        return c

                    lax.fori_loop(0, nss[t] // 16, fix, z)
                return carry

            row0 = wid * rpw + blk * RB
            pltpu.sync_copy(d_hbm.at[pl.ds(row0, RB)], dblk)
            lax.fori_loop(0, RB + 0 * wid, row, jnp.zeros((), I32))
            for t in range(3):
                pltpu.sync_copy(stgs[t], outs[t].at[pl.ds(row0, RB)])

    return k(dflat)


def _gather_call(tab, idx, C):
    R = idx.shape[0]
    G = 128
    K = 4
    rpw = R // NW
    nit = rpw // (G * K)
    idx2d = idx.reshape(R // G, G)
    mesh = plsc.VectorSubcoreMesh(core_axis_name="c", subcore_axis_name="s")

    @functools.partial(
        pl.kernel,
        out_type=jax.ShapeDtypeStruct((R, C), F32),
        mesh=mesh,
        compiler_params=pltpu.CompilerParams(use_tc_tiling_on_sc=False),
        scratch_types=[
            pltpu.VMEM((K, G), I32),
            pltpu.VMEM((K * G, C), F32),
            pltpu.SemaphoreType.DMA,
        ],
    )
    def k(tab_hbm, idx_hbm, out_hbm, idx_v, rows_v, sem):
        wid = lax.axis_index("s") * 2 + lax.axis_index("c")

        def chunk(i, c):
            base = wid * rpw + i * (G * K)
            pltpu.sync_copy(idx_hbm.at[pl.ds(base // G, K)], idx_v)
            cps = [
                pltpu.async_copy(tab_hbm.at[idx_v.at[j]],
                                 rows_v.at[pl.ds(j * G, G)], sem)
                for j in range(K)
            ]
            for cp in cps:
                cp.wait()
            pltpu.sync_copy(rows_v, out_hbm.at[pl.ds(base, G * K)])
            return c

        lax.fori_loop(0, nit + 0 * wid, chunk, jnp.zeros((), I32))

    return k(tab, idx2d)


def _pool_call(g3, nx8, w1, bias1, w2, bias2, w3, bias3, sa1_mode, P):
    BS, ns, C = g3.shape
    c3 = w3.shape[1]

    def body(g_ref, nx_ref, w1_ref, b1_ref, w2_ref, b2_ref, w3_ref, b3_ref,
             o_ref):
        g = g_ref[...]
        nx = nx_ref[...]
        if sa1_mode:
            gr = (g - nx[:, None, :]).reshape(P * ns, C)
            h = jnp.maximum(
                _mxdot(gr, w1_ref[...])
                + b1_ref[...], 0.0)
        else:
            v = _mxdot(nx, w1_ref[...])
            h1 = jnp.maximum(g - v[:, None, :] + b1_ref[...][None], 0.0)
            h = h1.reshape(P * ns, C)
        h = jnp.maximum(
            _mxdot(h, w2_ref[...]) + b2_ref[...],
            0.0)
        h = jnp.maximum(
            _mxdot(h, w3_ref[...]) + b3_ref[...],
            0.0)
        o_ref[...] = jnp.max(h.reshape(P, ns, c3), axis=1)

    grid = BS // P
    full = lambda shp: pl.BlockSpec(shp, lambda i: tuple(0 for _ in shp))
    return pl.pallas_call(
        body,
        grid=(grid,),
        in_specs=[
            pl.BlockSpec((P, ns, C), lambda i: (i, 0, 0)),
            pl.BlockSpec((P, 8), lambda i: (i, 0)),
            full(w1.shape), full(bias1.shape), full(w2.shape),
            full(bias2.shape), full(w3.shape), full(bias3.shape),
        ],
        out_specs=pl.BlockSpec((P, c3), lambda i: (i, 0)),
        out_shape=jax.ShapeDtypeStruct((BS, c3), F32),
    )(g3, nx8, w1, bias1, w2, bias2, w3, bias3)


def _u2_call(xpad, w1, w2, w3):
    def body(x_ref, wa_ref, wb_ref, wc_ref, oa_ref, ob_ref, oc_ref):
        xx = x_ref[...]
        oa_ref[...] = _mxdot(xx, wa_ref[...])
        ob_ref[...] = _mxdot(xx, wb_ref[...])
        oc_ref[...] = _mxdot(xx, wc_ref[...])

    n = xpad.shape[0]
    return pl.pallas_call(
        body,
        out_shape=[
            jax.ShapeDtypeStruct((n, w1.shape[1]), F32),
            jax.ShapeDtypeStruct((n, w2.shape[1]), F32),
            jax.ShapeDtypeStruct((n, w3.shape[1]), F32),
        ],
    )(xpad, w1, w2, w3)


def _tail_call(xp, weights):
    def body(x_ref, w1_ref, b1_ref, w2_ref, b2_ref, w3_ref, b3_ref,
             wl1_ref, bl1_ref, wl2_ref, bl2_ref, wd1_ref, bd1_ref,
             wd2_ref, bd2_ref, wd3_ref, bd3_ref, o_ref):
        h = jnp.maximum(
            _mxdot(x_ref[...], w1_ref[...])
            + b1_ref[...], 0.0)
        h = jnp.maximum(
            _mxdot(h, w2_ref[...]) + b2_ref[...],
            0.0)
        h = jnp.maximum(
            _mxdot(h, w3_ref[...]) + b3_ref[...],
            0.0)
        t = jnp.max(h.reshape(B, 128, 1024), axis=1)
        h = jnp.maximum(
            _mxdot(t, wl1_ref[...])
            + bl1_ref[...], 0.0)
        e = _mxdot(h, wl2_ref[...]) + bl2_ref[...]
        h = jnp.maximum(
            _mxdot(e, wd1_ref[...])
            + bd1_ref[...], 0.0)
        h = jnp.maximum(
            _mxdot(h, wd2_ref[...])
            + bd2_ref[...], 0.0)
        o_ref[...] = (_mxdot(h, wd3_ref[...])
                      + bd3_ref[...])

    return pl.pallas_call(
        body,
        out_shape=jax.ShapeDtypeStruct((B, 3 * NPTS), F32),
    )(xp, *weights)


def _row_bias(b):
    return b.reshape(1, -1)


def kernel(x, params):
    xt = x.transpose(0, 2, 1)

    ox, oy, oz = _fps_call(xt[:, 0], xt[:, 1], xt[:, 2], 512)
    nx1 = jnp.stack([ox, oy, oz], axis=-1)
    a8 = jnp.pad(nx1, ((0, 0), (0, 0), (0, 5)))
    t8 = jnp.pad(xt, ((0, 0), (0, 5), (0, 0)))
    d1 = _sqdist_call(a8, t8).reshape(B * 512, NPTS)
    i1, i2, i3 = _ballq_call(d1, 512, NPTS, (0.1, 0.2, 0.4), (16, 32, 128))

    xyz_tab = jnp.pad(x.reshape(B * NPTS, 3), ((0, 0), (0, 5)))
    nx8flat = a8.reshape(B * 512, 8)
    outs1 = []
    for idx, ns, br, P in zip((i1, i2, i3), (16, 32, 128), params['sa1'],
                              (256, 128, 32)):
        gfl = _gather_call(xyz_tab, idx.reshape(-1), 8)
        g3 = gfl.reshape(B * 512, ns, 8)
        w1p = jnp.pad(br[0]['W'], ((0, 5), (0, 0)))
        pooled = _pool_call(
            g3, nx8flat, w1p, _row_bias(br[0]['b']),
            br[1]['W'], _row_bias(br[1]['b']),
            br[2]['W'], _row_bias(br[2]['b']),
            sa1_mode=True, P=P)
        outs1.append(pooled)
    l1_points = jnp.concatenate(outs1, axis=-1)

    xt2 = nx1.transpose(0, 2, 1)
    ox2, oy2, oz2 = _fps_call(xt2[:, 0], xt2[:, 1], xt2[:, 2], 128)
    nx2 = jnp.stack([ox2, oy2, oz2], axis=-1)
    a8_2 = jnp.pad(nx2, ((0, 0), (0, 0), (0, 5)))
    t8_2 = jnp.pad(xt2, ((0, 0), (0, 5), (0, 0)))
    d2 = _sqdist_call(a8_2, t8_2).reshape(B * 128, 512)
    j1, j2, j3 = _ballq_call(d2, 128, 512, (0.2, 0.4, 0.8), (32, 64, 128))

    cat2 = jnp.concatenate([l1_points.reshape(B, 512, 320), nx1], axis=-1)
    xpad = jnp.pad(cat2.reshape(B * 512, 323), ((0, 0), (0, 61)))
    w1s = [jnp.pad(br[0]['W'], ((0, 61), (0, 0))) for br in params['sa2']]
    u1, u2, u3 = _u2_call(xpad, *w1s)

    nx8f2 = a8_2.reshape(B * 128, 8)
    outs2 = []
    for idx, ns, br, u, P in zip((j1, j2, j3), (32, 64, 128), params['sa2'],
                                 (u1, u2, u3), (128, 64, 32)):
        c1 = br[0]['W'].shape[1]
        gfl = _gather_call(u, idx.reshape(-1), c1)
        g3 = gfl.reshape(B * 128, ns, c1)
        w1v = jnp.pad(br[0]['W'][320:, :], ((0, 5), (0, 0)))
        pooled = _pool_call(
            g3, nx8f2, w1v, _row_bias(br[0]['b']),
            br[1]['W'], _row_bias(br[1]['b']),
            br[2]['W'], _row_bias(br[2]['b']),
            sa1_mode=False, P=P)
        outs2.append(pooled)
    l2_points = jnp.concatenate(outs2, axis=-1).reshape(B, 128, 640)

    l2cat = jnp.concatenate([nx2, l2_points], axis=-1)
    xp = jnp.pad(l2cat.reshape(B * 128, 643), ((0, 0), (0, 125)))
    sa3 = params['sa3']
    weights = [
        jnp.pad(sa3[0]['W'], ((0, 125), (0, 0))), _row_bias(sa3[0]['b']),
        sa3[1]['W'], _row_bias(sa3[1]['b']),
        sa3[2]['W'], _row_bias(sa3[2]['b']),
        params['lin1']['W'], _row_bias(params['lin1']['b']),
        params['lin2']['W'], _row_bias(params['lin2']['b']),
        params['dec'][0]['W'], _row_bias(params['dec'][0]['b']),
        params['dec'][1]['W'], _row_bias(params['dec'][1]['b']),
        params['dec'][2]['W'], _row_bias(params['dec'][2]['b']),
    ]
    out = _tail_call(xp, weights)
    return out.reshape(B, 3, NPTS).transpose(0, 2, 1)
